# Initial kernel scaffold; baseline (speedup 1.0000x reference)
import functools

import jax
import jax.numpy as jnp
from jax import lax
from jax.experimental import pallas as pl
from jax.experimental.pallas import tpu as pltpu
from jax.experimental.pallas import tpu_sc as plsc

N = 10000
E = 320000
D = 128

NC = 2
NS = 16
NW = NC * NS
EPW = E // NW
WIN = 80
NWIN = EPW // WIN
RPT = N // NS
ZCH = 125

_MESH = plsc.VectorSubcoreMesh(core_axis_name="c", subcore_axis_name="s")


def _zero_fill(buf, rows, cols):
    @pl.loop(0, rows)
    def _(r):
        for cc in range(0, cols, 16):
            buf.at[pl.ds(r, 1), pl.ds(cc, 16)][...] = jnp.zeros((1, 16), jnp.float32)


@functools.partial(
    pl.kernel,
    out_type=jax.ShapeDtypeStruct((NC, 2, N, 16), jnp.float32),
    mesh=_MESH,
    scratch_types=[
        pltpu.VMEM((NWIN, WIN), jnp.int32),
        pltpu.VMEM((NWIN, WIN), jnp.int32),
        pltpu.VMEM((WIN, 16), jnp.float32),
        pltpu.VMEM((ZCH, 16), jnp.float32),
        pltpu.VMEM_SHARED((N, 16), jnp.float32),
        pltpu.VMEM_SHARED((N, 16), jnp.float32),
    ],
)
def _degrees(src_hbm, dst_hbm, out_hbm, src_v, dst_v, ones_v, zbuf,
             acc_src, acc_dst):
    c = lax.axis_index("c")
    s = lax.axis_index("s")
    wid = c * NS + s
    pltpu.sync_copy(src_hbm.at[wid], src_v)
    pltpu.sync_copy(dst_hbm.at[wid], dst_v)

    @pl.loop(0, WIN)
    def _(r):
        ones_v.at[pl.ds(r, 1), pl.ds(0, 16)][...] = jnp.ones((1, 16), jnp.float32)

    _zero_fill(zbuf, ZCH, 16)

    @pl.loop(0, RPT, step=ZCH)
    def _(r0):
        row = s * RPT + r0
        pltpu.sync_copy(zbuf, acc_src.at[pl.ds(row, ZCH)])
        pltpu.sync_copy(zbuf, acc_dst.at[pl.ds(row, ZCH)])

    plsc.subcore_barrier()

    @pl.loop(0, NWIN)
    def _(w):
        pltpu.sync_copy(ones_v, acc_src.at[src_v.at[w]], add=True)
        pltpu.sync_copy(ones_v, acc_dst.at[dst_v.at[w]], add=True)

    plsc.subcore_barrier()

    @pl.loop(0, RPT, step=ZCH)
    def _(r0):
        row = s * RPT + r0
        pltpu.sync_copy(acc_src.at[pl.ds(row, ZCH)], zbuf)
        pltpu.sync_copy(zbuf, out_hbm.at[c, 0, pl.ds(row, ZCH)])
        pltpu.sync_copy(acc_dst.at[pl.ds(row, ZCH)], zbuf)
        pltpu.sync_copy(zbuf, out_hbm.at[c, 1, pl.ds(row, ZCH)])


@functools.partial(
    pl.kernel,
    out_type=jax.ShapeDtypeStruct((NC, N, D), jnp.float32),
    mesh=_MESH,
    scratch_types=[
        pltpu.VMEM((NWIN, WIN), jnp.int32),
        pltpu.VMEM((NWIN, WIN), jnp.int32),
        pltpu.VMEM((WIN, D), jnp.float32),
        pltpu.VMEM((ZCH, D), jnp.float32),
        pltpu.VMEM_SHARED((N, D), jnp.float32),
        pltpu.SemaphoreType.DMA,
    ],
)
def _aggregate(src_hbm, dst_hbm, xs_hbm, out_hbm, src_v, dst_v, rows_v, zbuf,
               acc, sem):
    c = lax.axis_index("c")
    s = lax.axis_index("s")
    wid = c * NS + s
    pltpu.sync_copy(src_hbm.at[wid], src_v)
    pltpu.sync_copy(dst_hbm.at[wid], dst_v)

    _zero_fill(zbuf, ZCH, D)

    @pl.loop(0, RPT, step=ZCH)
    def _(r0):
        pltpu.sync_copy(zbuf, acc.at[pl.ds(s * RPT + r0, ZCH)])

    plsc.subcore_barrier()

    @pl.loop(0, NWIN)
    def _(w):
        pltpu.async_copy(xs_hbm.at[src_v.at[w]], rows_v, sem).wait()
        pltpu.sync_copy(rows_v, acc.at[dst_v.at[w]], add=True)

    plsc.subcore_barrier()

    @pl.loop(0, RPT, step=ZCH)
    def _(r0):
        row = s * RPT + r0
        pltpu.sync_copy(acc.at[pl.ds(row, ZCH)], zbuf)
        pltpu.sync_copy(zbuf, out_hbm.at[c, pl.ds(row, ZCH)])


BN = 2000


def _prescale_body(x_ref, do0, do1, di0, di1, xs_ref, ns_ref, nd_ref):
    ns = lax.rsqrt(jnp.maximum(do0[...] + do1[...], 1.0))
    nd = lax.rsqrt(jnp.maximum(di0[...] + di1[...], 1.0))
    xs_ref[...] = x_ref[...] * ns
    ns_ref[...] = ns
    nd_ref[...] = nd


def _prescale(x, do0, do1, di0, di1):
    col = pl.BlockSpec((BN, 1), lambda i: (i, 0))
    mat = pl.BlockSpec((BN, D), lambda i: (i, 0))
    return pl.pallas_call(
        _prescale_body,
        grid=(N // BN,),
        in_specs=[mat, col, col, col, col],
        out_specs=[mat, col, col],
        out_shape=[
            jax.ShapeDtypeStruct((N, D), jnp.float32),
            jax.ShapeDtypeStruct((N, 1), jnp.float32),
            jax.ShapeDtypeStruct((N, 1), jnp.float32),
        ],
    )(x, do0, do1, di0, di1)


def _layer_body(m0, m1, nd, ns, w_ref, b_ref, out_ref, *, relu_scale):
    m = (m0[...] + m1[...]) * nd[...]
    y = jnp.dot(m, w_ref[...], precision=lax.Precision.HIGHEST,
                preferred_element_type=jnp.float32) + b_ref[...]
    if relu_scale:
        y = jnp.maximum(y, 0.0) * ns[...]
    out_ref[...] = y


def _layer(m0, m1, nd, ns, w, b, relu_scale):
    col = pl.BlockSpec((BN, 1), lambda i: (i, 0))
    mat = pl.BlockSpec((BN, D), lambda i: (i, 0))
    return pl.pallas_call(
        functools.partial(_layer_body, relu_scale=relu_scale),
        grid=(N // BN,),
        in_specs=[mat, mat, col, col,
                  pl.BlockSpec((D, D), lambda i: (0, 0)),
                  pl.BlockSpec((1, D), lambda i: (0, 0))],
        out_specs=mat,
        out_shape=jax.ShapeDtypeStruct((N, D), jnp.float32),
    )(m0, m1, nd, ns, w, b)


def kernel(x, edge_index, W1, b1, W2, b2):
    src = edge_index[0].reshape(NW, NWIN, WIN)
    dst = edge_index[1].reshape(NW, NWIN, WIN)

    degs = _degrees(src, dst)
    do0 = degs[0, 0, :, 0:1]
    do1 = degs[1, 0, :, 0:1]
    di0 = degs[0, 1, :, 0:1]
    di1 = degs[1, 1, :, 0:1]

    xs, ns, nd = _prescale(x, do0, do1, di0, di1)

    agg1 = _aggregate(src, dst, xs)
    hs = _layer(agg1[0], agg1[1], nd, ns, W1, b1.reshape(1, D), True)

    agg2 = _aggregate(src, dst, hs)
    out = _layer(agg2[0], agg2[1], nd, ns, W2, b2.reshape(1, D), False)
    return out


# speedup vs baseline: 6.5464x; 6.5464x over previous
import dataclasses
import functools

import jax
import jax.numpy as jnp
from jax import lax
from jax.experimental import pallas as pl
from jax.experimental.pallas import tpu as pltpu
from jax.experimental.pallas import tpu_sc as plsc

N = 10000
E = 320000
D = 128

NC = 2
NS = 16
NW = NC * NS
EPW = E // NW
WIN = 80
NWIN = EPW // WIN
CAP = EPW + WIN + 16

PHN = 5000
ACCR = 5120
ART = ACCR // NS
NTRASH = ACCR - PHN

_SC_PARAMS = pltpu.CompilerParams()
if "needs_layout_passes" in pltpu.CompilerParams.__dataclass_fields__:
    _SC_PARAMS = dataclasses.replace(_SC_PARAMS, needs_layout_passes=False)

_MESH = plsc.VectorSubcoreMesh(core_axis_name="c", subcore_axis_name="s")


@functools.partial(
    pl.kernel,
    out_type=jax.ShapeDtypeStruct((2 * NW * N,), jnp.float32),
    mesh=_MESH,
    compiler_params=_SC_PARAMS,
    scratch_types=[
        pltpu.VMEM((EPW,), jnp.int32),
        pltpu.VMEM((EPW,), jnp.int32),
        pltpu.VMEM((N,), jnp.float32),
        pltpu.VMEM((N,), jnp.float32),
    ],
)
def _degrees(src_hbm, dst_hbm, out_hbm, src_all, dst_all, cnt_s, cnt_d):
    c = lax.axis_index("c")
    s = lax.axis_index("s")
    wid = c * NS + s
    pltpu.sync_copy(src_hbm.at[pl.ds(wid * EPW, EPW)], src_all)
    pltpu.sync_copy(dst_hbm.at[pl.ds(wid * EPW, EPW)], dst_all)

    zeros = jnp.zeros((16,), jnp.float32)

    @pl.loop(0, N, step=16)
    def _(i):
        cnt_s.at[pl.ds(i, 16)][...] = zeros
        cnt_d.at[pl.ds(i, 16)][...] = zeros

    ones = jnp.ones((16,), jnp.float32)

    @pl.loop(0, EPW, step=16)
    def _(i):
        sv = src_all.at[pl.ds(i, 16)][...]
        plsc.addupdate_scatter(cnt_s, [sv], ones)
        dv = dst_all.at[pl.ds(i, 16)][...]
        plsc.addupdate_scatter(cnt_d, [dv], ones)

    pltpu.sync_copy(cnt_s, out_hbm.at[pl.ds(wid * N, N)])
    pltpu.sync_copy(cnt_d, out_hbm.at[pl.ds((NW + wid) * N, N)])


@functools.partial(
    pl.kernel,
    out_type=jax.ShapeDtypeStruct((NC, 2, ACCR, D), jnp.float32),
    mesh=_MESH,
    compiler_params=_SC_PARAMS,
    scratch_types=[
        pltpu.VMEM((EPW,), jnp.int32),
        pltpu.VMEM((EPW,), jnp.int32),
        pltpu.VMEM((CAP,), jnp.int32),
        pltpu.VMEM((CAP,), jnp.int32),
        pltpu.VMEM((CAP,), jnp.int32),
        pltpu.VMEM((CAP,), jnp.int32),
        pltpu.VMEM((WIN,), jnp.int32),
        pltpu.VMEM((WIN,), jnp.int32),
        pltpu.VMEM((WIN, D), jnp.float32),
        pltpu.VMEM((64, D), jnp.float32),
        pltpu.VMEM((64, D), jnp.float32),
        pltpu.VMEM_SHARED((ACCR, D), jnp.float32),
        pltpu.SMEM((2,), jnp.int32),
        pltpu.SemaphoreType.DMA,
    ],
)
def _aggregate(src_hbm, dst_hbm, xs_hbm, out_hbm, src_all, dst_all,
               sl0, dl0, sl1, dl1, swin, dwin, rows_v, zbuf, wbuf, acc, off, sem):
    c = lax.axis_index("c")
    s = lax.axis_index("s")
    wid = c * NS + s
    pltpu.sync_copy(src_hbm.at[pl.ds(wid * EPW, EPW)], src_all)
    pltpu.sync_copy(dst_hbm.at[pl.ds(wid * EPW, EPW)], dst_all)

    @pl.loop(0, 64)
    def _(r):
        for cc in range(0, D, 16):
            zbuf.at[r, pl.ds(cc, 16)][...] = jnp.zeros((16,), jnp.float32)

    off[0] = 0
    off[1] = 0

    @pl.loop(0, EPW, step=16)
    def _(i):
        dv = dst_all.at[pl.ds(i, 16)][...]
        sv = src_all.at[pl.ds(i, 16)][...]
        m0 = dv < PHN
        n0 = off[0]
        plsc.store_compressed(dl0.at[pl.ds(n0, 16)], dv, mask=m0)
        plsc.store_compressed(sl0.at[pl.ds(n0, 16)], sv, mask=m0)
        off[0] = n0 + jnp.sum(m0.astype(jnp.int32))
        m1 = dv >= PHN
        n1 = off[1]
        plsc.store_compressed(dl1.at[pl.ds(n1, 16)], dv - PHN, mask=m1)
        plsc.store_compressed(sl1.at[pl.ds(n1, 16)], sv, mask=m1)
        off[1] = n1 + jnp.sum(m1.astype(jnp.int32))

    lane = lax.iota(jnp.int32, 16)
    full = lane >= 0
    trash_dst = PHN + ((wid * 16 + lane) % NTRASH)
    trash_src = (wid * 313 + lane * 41) % N
    for p, (slp, dlp) in enumerate(((sl0, dl0), (sl1, dl1))):
        n = off[p]

        @pl.loop(0, WIN + 16, step=16)
        def _(j):
            plsc.store_compressed(dlp.at[pl.ds(n + j, 16)], trash_dst, mask=full)
            plsc.store_compressed(slp.at[pl.ds(n + j, 16)], trash_src, mask=full)

    for p, (slp, dlp) in enumerate(((sl0, dl0), (sl1, dl1))):
        @pl.loop(0, ART, step=64)
        def _(r0):
            pltpu.sync_copy(zbuf, acc.at[pl.ds(s * ART + r0, 64)])

        plsc.subcore_barrier()

        nwin = (off[p] + WIN - 1) // WIN

        @pl.loop(0, nwin)
        def _(w):
            for v in range(0, WIN, 16):
                swin.at[pl.ds(v, 16)][...] = slp.at[pl.ds(w * WIN + v, 16)][...]
                dwin.at[pl.ds(v, 16)][...] = dlp.at[pl.ds(w * WIN + v, 16)][...]
            pltpu.async_copy(xs_hbm.at[swin], rows_v, sem).wait()
            pltpu.sync_copy(rows_v, acc.at[dwin], add=True)

        plsc.subcore_barrier()

        @pl.loop(0, ART, step=64)
        def _(r0):
            row = s * ART + r0
            pltpu.sync_copy(acc.at[pl.ds(row, 64)], wbuf)
            pltpu.sync_copy(wbuf, out_hbm.at[c, p, pl.ds(row, 64)])

        plsc.subcore_barrier()


BN = 2000
BN2 = 1000


def _prescale_body(x_ref, dop, dip, xs_ref, ns_ref, nd_ref):
    dout = jnp.sum(dop[...], axis=1, keepdims=True)
    din = jnp.sum(dip[...], axis=1, keepdims=True)
    ns = lax.rsqrt(jnp.maximum(dout, 1.0))
    nd = lax.rsqrt(jnp.maximum(din, 1.0))
    xs_ref[...] = x_ref[...] * ns
    ns_ref[...] = ns
    nd_ref[...] = nd


def _prescale(x, dop_t, dip_t):
    part = pl.BlockSpec((BN, NW), lambda i: (i, 0))
    col = pl.BlockSpec((BN, 1), lambda i: (i, 0))
    mat = pl.BlockSpec((BN, D), lambda i: (i, 0))
    return pl.pallas_call(
        _prescale_body,
        grid=(N // BN,),
        in_specs=[mat, part, part],
        out_specs=[mat, col, col],
        out_shape=[
            jax.ShapeDtypeStruct((N, D), jnp.float32),
            jax.ShapeDtypeStruct((N, 1), jnp.float32),
            jax.ShapeDtypeStruct((N, 1), jnp.float32),
        ],
    )(x, dop_t, dip_t)


def _layer_body(m0, m1, nd, ns, w_ref, b_ref, out_ref, *, relu_scale):
    m = (m0[0] + m1[0]) * nd[...]
    y = jnp.dot(m, w_ref[...], precision=lax.Precision.HIGHEST,
                preferred_element_type=jnp.float32) + b_ref[...]
    if relu_scale:
        y = jnp.maximum(y, 0.0) * ns[...]
    out_ref[...] = y


def _layer(m0, m1, nd, ns, w, b, relu_scale):
    part = pl.BlockSpec((1, BN2, D), lambda p, i: (p, i, 0))
    col = pl.BlockSpec((BN2, 1), lambda p, i: (p * (PHN // BN2) + i, 0))
    return pl.pallas_call(
        functools.partial(_layer_body, relu_scale=relu_scale),
        grid=(2, PHN // BN2),
        in_specs=[part, part, col, col,
                  pl.BlockSpec((D, D), lambda p, i: (0, 0)),
                  pl.BlockSpec((1, D), lambda p, i: (0, 0))],
        out_specs=pl.BlockSpec((BN2, D), lambda p, i: (p * (PHN // BN2) + i, 0)),
        out_shape=jax.ShapeDtypeStruct((N, D), jnp.float32),
    )(m0, m1, nd, ns, w, b)


def kernel(x, edge_index, W1, b1, W2, b2):
    src = edge_index[0]
    dst = edge_index[1]

    degs = _degrees(src, dst).reshape(2, NW, N)
    dop_t = jnp.transpose(degs[0])
    dip_t = jnp.transpose(degs[1])

    xs, ns, nd = _prescale(x, dop_t, dip_t)

    agg = _aggregate(src, dst, xs)
    hs = _layer(agg[0], agg[1], nd, ns, W1, b1.reshape(1, D), True)

    agg = _aggregate(src, dst, hs)
    out = _layer(agg[0], agg[1], nd, ns, W2, b2.reshape(1, D), False)
    return out
